# baseline (speedup 1.0000x reference)
import numpy as np
import jax
import jax.numpy as jnp
from jax.experimental import pallas as pl
from jax.experimental.pallas import tpu as pltpu

_EPS = 1e-5

_CIN, _HIN, _WIN = 3, 32, 32
_KS = 5
_CO1, _HC1, _WC1, _HP1, _WP1 = 6, 28, 28, 14, 14
_CO2, _HC2, _WC2, _HP2, _WP2 = 16, 10, 10, 5, 5
_LP = 192
_LP2 = 256
_NQ = 512
_LQ1 = _WP1 * _CO1
_LQ2 = _WP2 * _CO2
_HR2 = 8

_CP = pltpu.CompilerParams(
    dimension_semantics=("arbitrary",),
    vmem_limit_bytes=64 * 1024 * 1024,
)


def _qpool(y):
    q0, q1 = y[:, 0:128], y[:, 128:256]
    q2, q3 = y[:, 256:384], y[:, 384:512]
    pm = jnp.maximum(jnp.maximum(q0, q1), jnp.maximum(q2, q3))
    pn = jnp.minimum(jnp.minimum(q0, q1), jnp.minimum(q2, q3))
    return pm, pn


def _stage1(x_ref, w_ref, b_ref, sg_ref, pool_ref, st_ref):
    bt = x_ref.shape[0]
    rows = bt * 8
    ps = [x_ref[:, ci].reshape(rows, 128).astype(jnp.bfloat16)
          for ci in range(_CIN)]
    xc = jnp.concatenate(ps + [jnp.roll(p, -1, axis=0) for p in ps], axis=1)
    y = jnp.dot(xc, w_ref[...], preferred_element_type=jnp.float32) + b_ref[...]
    ri = jax.lax.broadcasted_iota(jnp.int32, (rows, 1), 0)
    ym = y * ((ri & 7) < 7).astype(jnp.float32)
    s = jnp.sum(ym, axis=0, keepdims=True)
    q = jnp.sum(ym * y, axis=0, keepdims=True)
    st_ref[...] = jnp.concatenate([s, q], axis=0).reshape(1, 2, 2 * _NQ)
    sg = sg_ref[...] >= 0.5
    pm0, pn0 = _qpool(y[:, 0:_NQ])
    pm1, pn1 = _qpool(y[:, _NQ:2 * _NQ])
    pool_ref[:, 0:128] = jnp.where(sg, pm0, pn0).astype(jnp.bfloat16)
    pool_ref[:, 128:256] = jnp.where(sg, pm1, pn1).astype(jnp.bfloat16)


def _bn_lanes(sti_ref, f_ref, g_ref, be_ref, n):
    st = sti_ref[...]
    mom = jnp.concatenate([jnp.sum(st[:, 0, :], axis=0, keepdims=True),
                           jnp.sum(st[:, 1, :], axis=0, keepdims=True)], 0)
    fol = jnp.dot(mom, f_ref[...], preferred_element_type=jnp.float32)
    mean = fol[0:1] * (1.0 / n)
    var = fol[1:2] * (1.0 / n) - mean * mean
    sc = g_ref[...] * jax.lax.rsqrt(var + _EPS)
    return sc, be_ref[...] - mean * sc


def _stage2(p_ref, sti_ref, f_ref, g_ref, be_ref, w_ref, b_ref, sg_ref,
            q_ref, st_ref, *, n):
    rows = p_ref.shape[0]
    sc, sh = _bn_lanes(sti_ref, f_ref, g_ref, be_ref, n)
    x2 = jnp.maximum(p_ref[...].astype(jnp.float32) * sc
                     + sh, 0.0).astype(jnp.bfloat16)
    acc = jnp.dot(x2, w_ref[0], preferred_element_type=jnp.float32)
    for t in range(1, 3):
        acc = acc + jnp.dot(jnp.roll(x2, -t, axis=0), w_ref[t],
                            preferred_element_type=jnp.float32)
    y = acc + b_ref[...]
    ri = jax.lax.broadcasted_iota(jnp.int32, (rows, 1), 0)
    ym = y * ((ri & (_HR2 - 1)) < _HP2).astype(jnp.float32)
    s = jnp.sum(ym, axis=0, keepdims=True)
    q = jnp.sum(ym * y, axis=0, keepdims=True)
    st_ref[...] = jnp.concatenate([s, q], axis=0).reshape(1, 2, _NQ)
    pm, pn = _qpool(y)
    q_ref[...] = jnp.where(sg_ref[...] >= 0.5, pm, pn).astype(jnp.bfloat16)


def _stage3(q_ref, sti_ref, f_ref, g_ref, be_ref, w3_ref, b3_ref, fold_ref,
            o_ref, *, n):
    rows = q_ref.shape[0]
    bt = rows // _HR2
    nc = fold_ref.shape[1]
    sc, sh = _bn_lanes(sti_ref, f_ref, g_ref, be_ref, n)
    xp = jnp.maximum(q_ref[...].astype(jnp.float32) * sc
                     + sh, 0.0).astype(jnp.bfloat16)
    z = jnp.maximum(
        jnp.dot(xp, w3_ref[...], preferred_element_type=jnp.float32)
        + b3_ref[...], 0.0)
    ri = jax.lax.broadcasted_iota(jnp.int32, (rows, 1), 0)
    zm = z * ((ri & (_HR2 - 1)) < _HP2).astype(jnp.float32)
    zs = jnp.sum(zm.reshape(bt, _HR2, _WP2 * nc), axis=1)
    o_ref[...] = jnp.dot(zs, fold_ref[...],
                         preferred_element_type=jnp.float32) * (1.0 / (_HP2 * _WP2))


def _shift_sel():
    t, r, d, h = np.meshgrid(np.arange(3), np.arange(2), np.arange(2),
                             np.arange(_KS), indexing="ij")
    return (h == 2 * t + r - d).astype(np.float32)


def _shift_sel4():
    t, r, u, d, h = np.meshgrid(np.arange(2), np.arange(4), np.arange(2),
                                np.arange(2), np.arange(_KS), indexing="ij")
    return (h == 4 * t + r - 2 * u - d).astype(np.float32)


def _col_sel(win, pairs):
    w, e, j, k = np.meshgrid(np.arange(win), np.arange(2), np.arange(pairs),
                             np.arange(_KS), indexing="ij")
    return (w == 2 * j + e + k).astype(np.float32)


def _pad_lanes(a, axis, to):
    pad = [(0, 0)] * a.ndim
    pad[axis] = (0, to - a.shape[axis])
    return jnp.pad(a, pad)


def _slabs1(w1):
    wp = jnp.transpose(w1, (2, 3, 1, 0)).astype(jnp.float32)
    a = jnp.asarray(_shift_sel4())
    bm = jnp.asarray(_col_sel(_WIN, _WP1))
    full = jnp.einsum("trudh,wejk,hkio->tirwudejo", a, bm, wp)
    full = full.reshape(2 * _CIN * 128, 8, _LQ1)
    return _pad_lanes(full, 2, 128).reshape(
        2 * _CIN * 128, 2 * _NQ).astype(jnp.bfloat16)


def _slabs2(w2):
    wp = jnp.transpose(w2, (2, 3, 1, 0)).astype(jnp.float32)
    a = jnp.asarray(_shift_sel())
    bm = jnp.asarray(_col_sel(_WP1, _WP2))
    full = jnp.einsum("trdh,jepk,hkio->trjidepo", a, bm, wp)
    full = full.reshape(3, 2, _LQ1, 4, _LQ2)
    full = _pad_lanes(full, 2, 128)
    full = _pad_lanes(full, 4, 128)
    return full.reshape(3, _LP2, _NQ).astype(jnp.bfloat16)


def _head_w(w3):
    nc = w3.shape[0]
    wm = jnp.transpose(w3.reshape(nc, _CO2)).astype(jnp.float32)
    eye = jnp.eye(_WP2, dtype=jnp.float32)
    full = jnp.einsum("pg,io->pigo", eye, wm).reshape(_LQ2, _WP2 * nc)
    return _pad_lanes(full, 0, 128).astype(jnp.bfloat16)


def _qbias(bv, c, vq, chunks):
    reps = vq // c
    q = _pad_lanes(jnp.tile(bv.astype(jnp.float32), reps), 0, 128)
    return jnp.tile(q, chunks).reshape(1, chunks * 128)


def _lane_vec(vc, c, vq, reps):
    v = _pad_lanes(jnp.tile(vc, vq // c), 0, 128)
    return jnp.tile(v, reps).reshape(1, reps * 128)


def _fold_mat(chunks, c, vq, reps):
    li = np.arange(chunks * 128)
    lo = np.arange(reps * 128)
    qi, qo = li % 128, lo % 128
    m = ((qi[:, None] < vq) & (qo[None, :] < vq) &
         ((qi[:, None] % c) == (qo[None, :] % c)))
    return jnp.asarray(m.astype(np.float32))


def _bn_fold(st, c, vq, n, gamma, beta):
    sv = st.reshape(st.shape[0], 2, -1, 128)[:, :, :, :vq]
    sv = sv.reshape(sv.shape[0], 2, -1, c)
    s = jnp.sum(sv[:, 0], axis=(0, 1))
    q = jnp.sum(sv[:, 1], axis=(0, 1))
    mean = s / n
    var = q / n - mean * mean
    sc_c = gamma * jax.lax.rsqrt(var + _EPS)
    sh_c = beta - mean * sc_c
    return sc_c, sh_c


def _tile(b):
    for t in (512, 256, 128, 64, 32, 16, 8):
        if b % t == 0:
            return t
    return b


def kernel(x, w1, b1, g1, be1, w2, b2, g2, be2, w3, b3):
    x = x.reshape(-1, _CIN, _HIN, _WIN)
    b = x.shape[0]
    nc = w3.shape[0]
    bt = _tile(b)
    nt = b // bt

    xm = x.reshape(b, _CIN, 8, 128)

    wb1 = _slabs1(w1)
    wb2 = _slabs2(w2)
    b1t = _qbias(b1, _CO1, _LQ1, 8)
    b2t = _qbias(b2, _CO2, _LQ2, 4)
    w3b = _head_w(w3)
    b3t = jnp.tile(b3, _WP2).reshape(1, _WP2 * nc).astype(jnp.float32)
    fold = jnp.asarray(np.tile(np.eye(nc, dtype=np.float32), (_WP2, 1)))
    sg1 = _lane_vec((g1 >= 0).astype(jnp.float32), _CO1, _LQ1, 1)
    sg2 = _lane_vec((g2 >= 0).astype(jnp.float32), _CO2, _LQ2, 1)
    f1 = _fold_mat(8, _CO1, _LQ1, 2)
    f2 = _fold_mat(4, _CO2, _LQ2, 1)
    g1l = _lane_vec(g1.astype(jnp.float32), _CO1, _LQ1, 2)
    be1l = _lane_vec(be1.astype(jnp.float32), _CO1, _LQ1, 2)
    g2l = _lane_vec(g2.astype(jnp.float32), _CO2, _LQ2, 1)
    be2l = _lane_vec(be2.astype(jnp.float32), _CO2, _LQ2, 1)

    pool1, st1 = pl.pallas_call(
        _stage1,
        grid=(nt,),
        in_specs=[pl.BlockSpec((bt, _CIN, 8, 128), lambda t: (t, 0, 0, 0)),
                  pl.BlockSpec((2 * _CIN * 128, 2 * _NQ), lambda t: (0, 0)),
                  pl.BlockSpec((1, 2 * _NQ), lambda t: (0, 0)),
                  pl.BlockSpec((1, 128), lambda t: (0, 0))],
        out_specs=[pl.BlockSpec((bt * 8, _LP2), lambda t: (t, 0)),
                   pl.BlockSpec((1, 2, 2 * _NQ), lambda t: (t, 0, 0))],
        out_shape=[jax.ShapeDtypeStruct((b * 8, _LP2), jnp.bfloat16),
                   jax.ShapeDtypeStruct((nt, 2, 2 * _NQ), jnp.float32)],
        compiler_params=_CP,
    )(xm, wb1, b1t, sg1)

    import functools as _ft
    pool2, st2 = pl.pallas_call(
        _ft.partial(_stage2, n=float(b * _HC1 * _WC1)),
        grid=(nt,),
        in_specs=[pl.BlockSpec((bt * 8, _LP2), lambda t: (t, 0)),
                  pl.BlockSpec((nt, 2, 2 * _NQ), lambda t: (0, 0, 0)),
                  pl.BlockSpec((2 * _NQ, _LP2), lambda t: (0, 0)),
                  pl.BlockSpec((1, _LP2), lambda t: (0, 0)),
                  pl.BlockSpec((1, _LP2), lambda t: (0, 0)),
                  pl.BlockSpec((3, _LP2, _NQ), lambda t: (0, 0, 0)),
                  pl.BlockSpec((1, _NQ), lambda t: (0, 0)),
                  pl.BlockSpec((1, 128), lambda t: (0, 0))],
        out_specs=[pl.BlockSpec((bt * _HR2, 128), lambda t: (t, 0)),
                   pl.BlockSpec((1, 2, _NQ), lambda t: (t, 0, 0))],
        out_shape=[jax.ShapeDtypeStruct((b * _HR2, 128), jnp.bfloat16),
                   jax.ShapeDtypeStruct((nt, 2, _NQ), jnp.float32)],
        compiler_params=_CP,
    )(pool1, st1, f1, g1l, be1l, wb2, b2t, sg2)

    out = pl.pallas_call(
        _ft.partial(_stage3, n=float(b * _HC2 * _WC2)),
        grid=(nt,),
        in_specs=[pl.BlockSpec((bt * _HR2, 128), lambda t: (t, 0)),
                  pl.BlockSpec((nt, 2, _NQ), lambda t: (0, 0, 0)),
                  pl.BlockSpec((_NQ, 128), lambda t: (0, 0)),
                  pl.BlockSpec((1, 128), lambda t: (0, 0)),
                  pl.BlockSpec((1, 128), lambda t: (0, 0)),
                  pl.BlockSpec((128, _WP2 * nc), lambda t: (0, 0)),
                  pl.BlockSpec((1, _WP2 * nc), lambda t: (0, 0)),
                  pl.BlockSpec((_WP2 * nc, nc), lambda t: (0, 0))],
        out_specs=pl.BlockSpec((bt, nc), lambda t: (t, 0)),
        out_shape=jax.ShapeDtypeStruct((b, nc), jnp.float32),
        compiler_params=_CP,
    )(pool2, st2, f2, g2l, be2l, w3b, b3t, fold)
    return out


# speedup vs baseline: 1.0023x; 1.0023x over previous
import numpy as np
import jax
import jax.numpy as jnp
from jax.experimental import pallas as pl
from jax.experimental.pallas import tpu as pltpu

_EPS = 1e-5

_CIN, _HIN, _WIN = 3, 32, 32
_KS = 5
_CO1, _HC1, _WC1, _HP1, _WP1 = 6, 28, 28, 14, 14
_CO2, _HC2, _WC2, _HP2, _WP2 = 16, 10, 10, 5, 5
_LP = 192
_LP2 = 256
_NQ = 512
_LQ1 = _WP1 * _CO1
_LQ2 = _WP2 * _CO2
_HR2 = 8

_CP = pltpu.CompilerParams(
    dimension_semantics=("arbitrary",),
    vmem_limit_bytes=64 * 1024 * 1024,
)


def _qpool(y):
    q0, q1 = y[:, 0:128], y[:, 128:256]
    q2, q3 = y[:, 256:384], y[:, 384:512]
    pm = jnp.maximum(jnp.maximum(q0, q1), jnp.maximum(q2, q3))
    pn = jnp.minimum(jnp.minimum(q0, q1), jnp.minimum(q2, q3))
    return pm, pn


def _stage1(x_ref, w_ref, b_ref, sg_ref, pool_ref, st_ref):
    bt = x_ref.shape[0]
    rows = bt * 8
    ps = [x_ref[:, ci].reshape(rows, 128).astype(jnp.bfloat16)
          for ci in range(_CIN)]
    xc = jnp.concatenate(ps + [jnp.roll(p, -1, axis=0) for p in ps], axis=1)
    y = jnp.dot(xc, w_ref[...], preferred_element_type=jnp.float32) + b_ref[...]
    ri = jax.lax.broadcasted_iota(jnp.int32, (rows, 1), 0)
    ym = y * ((ri & 7) < 7).astype(jnp.float32)
    s = jnp.sum(ym, axis=0, keepdims=True)
    q = jnp.sum(ym * y, axis=0, keepdims=True)
    st_ref[...] = jnp.concatenate([s, q], axis=0).reshape(1, 2, 2 * _NQ)
    sg = sg_ref[...] >= 0.5
    pm0, pn0 = _qpool(y[:, 0:_NQ])
    pm1, pn1 = _qpool(y[:, _NQ:2 * _NQ])
    pool_ref[:, 0:128] = jnp.where(sg, pm0, pn0).astype(jnp.bfloat16)
    pool_ref[:, 128:256] = jnp.where(sg, pm1, pn1).astype(jnp.bfloat16)


def _stage2(p_ref, sc_ref, sh_ref, w_ref, b_ref, sg_ref,
            q_ref, st_ref):
    rows = p_ref.shape[0]
    x2 = jnp.maximum(p_ref[...].astype(jnp.float32) * sc_ref[...]
                     + sh_ref[...], 0.0).astype(jnp.bfloat16)
    acc = jnp.dot(x2, w_ref[0], preferred_element_type=jnp.float32)
    for t in range(1, 3):
        acc = acc + jnp.dot(jnp.roll(x2, -t, axis=0), w_ref[t],
                            preferred_element_type=jnp.float32)
    y = acc + b_ref[...]
    ri = jax.lax.broadcasted_iota(jnp.int32, (rows, 1), 0)
    ym = y * ((ri & (_HR2 - 1)) < _HP2).astype(jnp.float32)
    s = jnp.sum(ym, axis=0, keepdims=True)
    q = jnp.sum(ym * y, axis=0, keepdims=True)
    st_ref[...] = jnp.concatenate([s, q], axis=0).reshape(1, 2, _NQ)
    pm, pn = _qpool(y)
    q_ref[...] = jnp.where(sg_ref[...] >= 0.5, pm, pn).astype(jnp.bfloat16)


def _stage3(q_ref, sc_ref, sh_ref, w3_ref, b3_ref, fold_ref,
            o_ref):
    rows = q_ref.shape[0]
    bt = rows // _HR2
    nc = fold_ref.shape[1]
    xp = jnp.maximum(q_ref[...].astype(jnp.float32) * sc_ref[...]
                     + sh_ref[...], 0.0).astype(jnp.bfloat16)
    z = jnp.maximum(
        jnp.dot(xp, w3_ref[...], preferred_element_type=jnp.float32)
        + b3_ref[...], 0.0)
    ri = jax.lax.broadcasted_iota(jnp.int32, (rows, 1), 0)
    zm = z * ((ri & (_HR2 - 1)) < _HP2).astype(jnp.float32)
    zs = jnp.sum(zm.reshape(bt, _HR2, _WP2 * nc), axis=1)
    o_ref[...] = jnp.dot(zs, fold_ref[...],
                         preferred_element_type=jnp.float32) * (1.0 / (_HP2 * _WP2))


def _shift_sel():
    t, r, d, h = np.meshgrid(np.arange(3), np.arange(2), np.arange(2),
                             np.arange(_KS), indexing="ij")
    return (h == 2 * t + r - d).astype(np.float32)


def _shift_sel4():
    t, r, u, d, h = np.meshgrid(np.arange(2), np.arange(4), np.arange(2),
                                np.arange(2), np.arange(_KS), indexing="ij")
    return (h == 4 * t + r - 2 * u - d).astype(np.float32)


def _col_sel(win, pairs):
    w, e, j, k = np.meshgrid(np.arange(win), np.arange(2), np.arange(pairs),
                             np.arange(_KS), indexing="ij")
    return (w == 2 * j + e + k).astype(np.float32)


def _pad_lanes(a, axis, to):
    pad = [(0, 0)] * a.ndim
    pad[axis] = (0, to - a.shape[axis])
    return jnp.pad(a, pad)


def _slabs1(w1):
    wp = jnp.transpose(w1, (2, 3, 1, 0)).astype(jnp.float32)
    a = jnp.asarray(_shift_sel4())
    bm = jnp.asarray(_col_sel(_WIN, _WP1))
    full = jnp.einsum("trudh,wejk,hkio->tirwudejo", a, bm, wp)
    full = full.reshape(2 * _CIN * 128, 8, _LQ1)
    return _pad_lanes(full, 2, 128).reshape(
        2 * _CIN * 128, 2 * _NQ).astype(jnp.bfloat16)


def _slabs2(w2):
    wp = jnp.transpose(w2, (2, 3, 1, 0)).astype(jnp.float32)
    a = jnp.asarray(_shift_sel())
    bm = jnp.asarray(_col_sel(_WP1, _WP2))
    full = jnp.einsum("trdh,jepk,hkio->trjidepo", a, bm, wp)
    full = full.reshape(3, 2, _LQ1, 4, _LQ2)
    full = _pad_lanes(full, 2, 128)
    full = _pad_lanes(full, 4, 128)
    return full.reshape(3, _LP2, _NQ).astype(jnp.bfloat16)


def _head_w(w3):
    nc = w3.shape[0]
    wm = jnp.transpose(w3.reshape(nc, _CO2)).astype(jnp.float32)
    eye = jnp.eye(_WP2, dtype=jnp.float32)
    full = jnp.einsum("pg,io->pigo", eye, wm).reshape(_LQ2, _WP2 * nc)
    return _pad_lanes(full, 0, 128).astype(jnp.bfloat16)


def _qbias(bv, c, vq, chunks):
    reps = vq // c
    q = _pad_lanes(jnp.tile(bv.astype(jnp.float32), reps), 0, 128)
    return jnp.tile(q, chunks).reshape(1, chunks * 128)


def _lane_vec(vc, c, vq, reps):
    v = _pad_lanes(jnp.tile(vc, vq // c), 0, 128)
    return jnp.tile(v, reps).reshape(1, reps * 128)


def _bn_fold(st, c, vq, n, gamma, beta):
    sv = st.reshape(st.shape[0], 2, -1, 128)[:, :, :, :vq]
    sv = sv.reshape(sv.shape[0], 2, -1, c)
    s = jnp.sum(sv[:, 0], axis=(0, 1))
    q = jnp.sum(sv[:, 1], axis=(0, 1))
    mean = s / n
    var = q / n - mean * mean
    sc_c = gamma * jax.lax.rsqrt(var + _EPS)
    sh_c = beta - mean * sc_c
    return sc_c, sh_c


def _tile(b):
    for t in (512, 256, 128, 64, 32, 16, 8):
        if b % t == 0:
            return t
    return b


def kernel(x, w1, b1, g1, be1, w2, b2, g2, be2, w3, b3):
    x = x.reshape(-1, _CIN, _HIN, _WIN)
    b = x.shape[0]
    nc = w3.shape[0]
    bt = _tile(b)
    nt = b // bt

    xm = x.reshape(b, _CIN, 8, 128)

    wb1 = _slabs1(w1)
    wb2 = _slabs2(w2)
    b1t = _qbias(b1, _CO1, _LQ1, 8)
    b2t = _qbias(b2, _CO2, _LQ2, 4)
    w3b = _head_w(w3)
    b3t = jnp.tile(b3, _WP2).reshape(1, _WP2 * nc).astype(jnp.float32)
    fold = jnp.asarray(np.tile(np.eye(nc, dtype=np.float32), (_WP2, 1)))
    sg1 = _lane_vec((g1 >= 0).astype(jnp.float32), _CO1, _LQ1, 1)
    sg2 = _lane_vec((g2 >= 0).astype(jnp.float32), _CO2, _LQ2, 1)

    pool1, st1 = pl.pallas_call(
        _stage1,
        grid=(nt,),
        in_specs=[pl.BlockSpec((bt, _CIN, 8, 128), lambda t: (t, 0, 0, 0)),
                  pl.BlockSpec((2 * _CIN * 128, 2 * _NQ), lambda t: (0, 0)),
                  pl.BlockSpec((1, 2 * _NQ), lambda t: (0, 0)),
                  pl.BlockSpec((1, 128), lambda t: (0, 0))],
        out_specs=[pl.BlockSpec((bt * 8, _LP2), lambda t: (t, 0)),
                   pl.BlockSpec((1, 2, 2 * _NQ), lambda t: (t, 0, 0))],
        out_shape=[jax.ShapeDtypeStruct((b * 8, _LP2), jnp.bfloat16),
                   jax.ShapeDtypeStruct((nt, 2, 2 * _NQ), jnp.float32)],
        compiler_params=_CP,
    )(xm, wb1, b1t, sg1)
    sc1c, sh1c = _bn_fold(st1, _CO1, _LQ1, b * _HC1 * _WC1, g1, be1)
    sc1 = _lane_vec(sc1c, _CO1, _LQ1, 2)
    sh1 = _lane_vec(sh1c, _CO1, _LQ1, 2)

    pool2, st2 = pl.pallas_call(
        _stage2,
        grid=(nt,),
        in_specs=[pl.BlockSpec((bt * 8, _LP2), lambda t: (t, 0)),
                  pl.BlockSpec((1, _LP2), lambda t: (0, 0)),
                  pl.BlockSpec((1, _LP2), lambda t: (0, 0)),
                  pl.BlockSpec((3, _LP2, _NQ), lambda t: (0, 0, 0)),
                  pl.BlockSpec((1, _NQ), lambda t: (0, 0)),
                  pl.BlockSpec((1, 128), lambda t: (0, 0))],
        out_specs=[pl.BlockSpec((bt * _HR2, 128), lambda t: (t, 0)),
                   pl.BlockSpec((1, 2, _NQ), lambda t: (t, 0, 0))],
        out_shape=[jax.ShapeDtypeStruct((b * _HR2, 128), jnp.bfloat16),
                   jax.ShapeDtypeStruct((nt, 2, _NQ), jnp.float32)],
        compiler_params=_CP,
    )(pool1, sc1, sh1, wb2, b2t, sg2)
    sc2c, sh2c = _bn_fold(st2, _CO2, _LQ2, b * _HC2 * _WC2, g2, be2)
    sc2 = _lane_vec(sc2c, _CO2, _LQ2, 1)
    sh2 = _lane_vec(sh2c, _CO2, _LQ2, 1)

    out = pl.pallas_call(
        _stage3,
        grid=(nt,),
        in_specs=[pl.BlockSpec((bt * _HR2, 128), lambda t: (t, 0)),
                  pl.BlockSpec((1, 128), lambda t: (0, 0)),
                  pl.BlockSpec((1, 128), lambda t: (0, 0)),
                  pl.BlockSpec((128, _WP2 * nc), lambda t: (0, 0)),
                  pl.BlockSpec((1, _WP2 * nc), lambda t: (0, 0)),
                  pl.BlockSpec((_WP2 * nc, nc), lambda t: (0, 0))],
        out_specs=pl.BlockSpec((bt, nc), lambda t: (t, 0)),
        out_shape=jax.ShapeDtypeStruct((b, nc), jnp.float32),
        compiler_params=_CP,
    )(pool2, sc2, sh2, w3b, b3t, fold)
    return out
